# baseline (device time: 23841 ns/iter reference)
import jax
import jax.numpy as jnp
from jax import lax
from jax.experimental import pallas as pl
from jax.experimental.pallas import tpu as pltpu

MESH = pl.DeviceIdType.MESH
N_CHUNK = 4


def kernel(Q, K, V):
    b, s, h, d = Q.shape
    bh = b * h
    csz = bh // N_CHUNK
    scale = d ** -0.5

    def to_heads(x):
        return jnp.reshape(jnp.transpose(x, (0, 2, 1, 3)), (bh, s, d)).astype(
            jnp.bfloat16
        )

    Qb = to_heads(Q * scale)
    Kb = to_heads(K)
    Vb = to_heads(V)

    def body(q_ref, k_ref, v_ref, o_ref, kr, vr, l1s, o1s, ssem, rsem):
        my_x = lax.axis_index("x")
        my_y = lax.axis_index("y")
        my_z = lax.axis_index("z")
        z_peer = (my_x, my_y, 1 - my_z)
        x_peer = (1 - my_x, my_y, my_z)
        y_peer = (my_x, 1 - my_y, my_z)
        is_kk = my_x == my_y
        fwd_lo = my_x == 0

        sl = [pl.ds(c * csz, csz) for c in range(N_CHUNK)]

        bar = pltpu.get_barrier_semaphore()
        for p in (z_peer, x_peer, y_peer):
            pl.semaphore_signal(bar, inc=1, device_id=p, device_id_type=MESH)
        pl.semaphore_wait(bar, 3)

        def start_z(src, dst, order):
            for p, cid in enumerate(order):
                pltpu.make_async_remote_copy(
                    src_ref=src.at[sl[cid]], dst_ref=dst.at[sl[cid]],
                    send_sem=ssem.at[p], recv_sem=rsem.at[p],
                    device_id=z_peer, device_id_type=MESH,
                ).start()

        for kk in (True, False):
            for flo in (True, False):
                @pl.when((is_kk == kk) & (fwd_lo == flo))
                def _(kk=kk, flo=flo):
                    src, dst = (k_ref, kr) if kk else (v_ref, vr)
                    start_z(src, dst, (0, 1, 2, 3) if flo else (2, 3, 0, 1))

        dn_qk = (((2,), (2,)), ((0,), (0,)))
        dn_pv = (((2,), (1,)), ((0,), (0,)))

        def attn_block(qv, kv, vv):
            st = lax.dot_general(qv, kv, dn_qk,
                                 preferred_element_type=jnp.float32)
            pt = jnp.exp(st)
            lt = jnp.sum(pt, axis=2)
            ot = lax.dot_general(pt.astype(jnp.bfloat16), vv, dn_pv,
                                 preferred_element_type=jnp.float32)
            return lt, ot

        qv = q_ref[...]
        l0, o0 = attn_block(qv, k_ref[...], v_ref[...])

        def wait_z(p):
            pltpu.make_async_remote_copy(
                src_ref=kr.at[sl[p]], dst_ref=kr.at[sl[p]],
                send_sem=ssem.at[p], recv_sem=rsem.at[p],
                device_id=z_peer, device_id_type=MESH,
            ).wait()

        for p in range(2):
            wait_z(p)
            for kk in (True, False):
                for flo in (True, False):
                    @pl.when((is_kk == kk) & (fwd_lo == flo))
                    def _(kk=kk, flo=flo, p=p):
                        buf = kr if kk else vr
                        cid = (0, 1)[p] if flo else (2, 3)[p]
                        for t, peer in enumerate((x_peer, y_peer)):
                            pltpu.make_async_remote_copy(
                                src_ref=buf.at[sl[cid]],
                                dst_ref=buf.at[sl[cid]],
                                send_sem=ssem.at[4 + 2 * p + t],
                                recv_sem=rsem.at[4 + cid],
                                device_id=peer, device_id_type=MESH,
                            ).start()

        def wait_fwd_recv(c):
            pltpu.make_async_remote_copy(
                src_ref=vr.at[sl[c]], dst_ref=vr.at[sl[c]],
                send_sem=ssem.at[0], recv_sem=rsem.at[4 + c],
                device_id=x_peer, device_id_type=MESH,
            ).wait_recv()

        def remote_half(h0):
            slc = pl.ds(h0, bh // 2)
            lt, ot = attn_block(qv[h0:h0 + bh // 2], kr[slc], vr[slc])
            l1s[pl.ds(h0, bh // 2)] = lt
            o1s[slc] = ot

        @pl.when(fwd_lo)
        def _():
            wait_fwd_recv(0)
            wait_fwd_recv(1)
            remote_half(0)
            wait_z(2)
            wait_z(3)
            wait_fwd_recv(2)
            wait_fwd_recv(3)
            remote_half(bh // 2)

        @pl.when(jnp.logical_not(fwd_lo))
        def _():
            wait_fwd_recv(2)
            wait_fwd_recv(3)
            remote_half(bh // 2)
            wait_z(2)
            wait_z(3)
            wait_fwd_recv(0)
            wait_fwd_recv(1)
            remote_half(0)

        for t in range(4):
            pltpu.make_async_remote_copy(
                src_ref=vr.at[sl[0]], dst_ref=vr.at[sl[0]],
                send_sem=ssem.at[4 + t], recv_sem=rsem.at[0],
                device_id=x_peer, device_id_type=MESH,
            ).wait_send()

        o_ref[...] = (o0 + o1s[...]) * (1.0 / (l0 + l1s[...]))[:, :, None]

    out = pl.pallas_call(
        body,
        out_shape=jax.ShapeDtypeStruct((bh, s, d), jnp.float32),
        in_specs=[pl.BlockSpec(memory_space=pltpu.VMEM)] * 3,
        out_specs=pl.BlockSpec(memory_space=pltpu.VMEM),
        scratch_shapes=[
            pltpu.VMEM((bh, s, d), jnp.bfloat16),
            pltpu.VMEM((bh, s, d), jnp.bfloat16),
            pltpu.VMEM((bh, s), jnp.float32),
            pltpu.VMEM((bh, s, d), jnp.float32),
            pltpu.SemaphoreType.DMA((8,)),
            pltpu.SemaphoreType.DMA((8,)),
        ],
        compiler_params=pltpu.CompilerParams(collective_id=0),
    )(Qb, Kb, Vb)

    return jnp.transpose(jnp.reshape(out, (b, h, s, d)), (0, 2, 1, 3))


# device time: 22639 ns/iter; 1.0531x vs baseline; 1.0531x over previous
import jax
import jax.numpy as jnp
from jax import lax
from jax.experimental import pallas as pl
from jax.experimental.pallas import tpu as pltpu

MESH = pl.DeviceIdType.MESH
N_CHUNK = 4


def kernel(Q, K, V):
    b, s, h, d = Q.shape
    bh = b * h
    csz = bh // N_CHUNK
    scale = d ** -0.5

    def to_heads(x):
        return jnp.reshape(
            jnp.transpose(x.astype(jnp.bfloat16), (0, 2, 1, 3)), (bh, s, d)
        )

    Qb = to_heads(Q * scale)
    Kb = to_heads(K)
    Vb = to_heads(V)

    def body(q_ref, k_ref, v_ref, o_ref, kr, vr, l1s, o1s, ssem, rsem):
        my_x = lax.axis_index("x")
        my_y = lax.axis_index("y")
        my_z = lax.axis_index("z")
        z_peer = (my_x, my_y, 1 - my_z)
        x_peer = (1 - my_x, my_y, my_z)
        y_peer = (my_x, 1 - my_y, my_z)
        is_kk = my_x == my_y
        fwd_lo = my_x == 0

        sl = [pl.ds(c * csz, csz) for c in range(N_CHUNK)]

        bar = pltpu.get_barrier_semaphore()
        for p in (z_peer, x_peer, y_peer):
            pl.semaphore_signal(bar, inc=1, device_id=p, device_id_type=MESH)
        pl.semaphore_wait(bar, 3)

        def start_z(src, dst, order):
            for p, cid in enumerate(order):
                pltpu.make_async_remote_copy(
                    src_ref=src.at[sl[cid]], dst_ref=dst.at[sl[cid]],
                    send_sem=ssem.at[p], recv_sem=rsem.at[p],
                    device_id=z_peer, device_id_type=MESH,
                ).start()

        for kk in (True, False):
            for flo in (True, False):
                @pl.when((is_kk == kk) & (fwd_lo == flo))
                def _(kk=kk, flo=flo):
                    src, dst = (k_ref, kr) if kk else (v_ref, vr)
                    start_z(src, dst, (0, 1, 2, 3) if flo else (2, 3, 0, 1))

        dn_qk = (((2,), (2,)), ((0,), (0,)))
        dn_pv = (((2,), (1,)), ((0,), (0,)))

        def attn_block(qv, kv, vv):
            st = lax.dot_general(qv, kv, dn_qk,
                                 preferred_element_type=jnp.float32)
            pt = jnp.exp(st)
            lt = jnp.sum(pt, axis=2)
            ot = lax.dot_general(pt.astype(jnp.bfloat16), vv, dn_pv,
                                 preferred_element_type=jnp.float32)
            return lt, ot

        qv = q_ref[...]
        l0, o0 = attn_block(qv, k_ref[...], v_ref[...])

        def wait_z(p):
            pltpu.make_async_remote_copy(
                src_ref=kr.at[sl[p]], dst_ref=kr.at[sl[p]],
                send_sem=ssem.at[p], recv_sem=rsem.at[p],
                device_id=z_peer, device_id_type=MESH,
            ).wait()

        for p in range(2):
            wait_z(p)
            for kk in (True, False):
                for flo in (True, False):
                    @pl.when((is_kk == kk) & (fwd_lo == flo))
                    def _(kk=kk, flo=flo, p=p):
                        buf = kr if kk else vr
                        cid = (0, 1)[p] if flo else (2, 3)[p]
                        for t, peer in enumerate((x_peer, y_peer)):
                            pltpu.make_async_remote_copy(
                                src_ref=buf.at[sl[cid]],
                                dst_ref=buf.at[sl[cid]],
                                send_sem=ssem.at[4 + 2 * p + t],
                                recv_sem=rsem.at[4 + cid],
                                device_id=peer, device_id_type=MESH,
                            ).start()

        def wait_fwd_recv(c):
            pltpu.make_async_remote_copy(
                src_ref=vr.at[sl[c]], dst_ref=vr.at[sl[c]],
                send_sem=ssem.at[0], recv_sem=rsem.at[4 + c],
                device_id=x_peer, device_id_type=MESH,
            ).wait_recv()

        def remote_half(h0):
            slc = pl.ds(h0, bh // 2)
            lt, ot = attn_block(qv[h0:h0 + bh // 2], kr[slc], vr[slc])
            l1s[pl.ds(h0, bh // 2)] = lt
            o1s[slc] = ot

        @pl.when(fwd_lo)
        def _():
            wait_fwd_recv(0)
            wait_fwd_recv(1)
            remote_half(0)
            wait_z(2)
            wait_z(3)
            wait_fwd_recv(2)
            wait_fwd_recv(3)
            remote_half(bh // 2)

        @pl.when(jnp.logical_not(fwd_lo))
        def _():
            wait_fwd_recv(2)
            wait_fwd_recv(3)
            remote_half(bh // 2)
            wait_z(2)
            wait_z(3)
            wait_fwd_recv(0)
            wait_fwd_recv(1)
            remote_half(0)

        for t in range(4):
            pltpu.make_async_remote_copy(
                src_ref=vr.at[sl[0]], dst_ref=vr.at[sl[0]],
                send_sem=ssem.at[4 + t], recv_sem=rsem.at[0],
                device_id=x_peer, device_id_type=MESH,
            ).wait_send()

        o_ref[...] = (o0 + o1s[...]) * (1.0 / (l0 + l1s[...]))[:, :, None]

    out = pl.pallas_call(
        body,
        out_shape=jax.ShapeDtypeStruct((bh, s, d), jnp.float32),
        in_specs=[pl.BlockSpec(memory_space=pltpu.VMEM)] * 3,
        out_specs=pl.BlockSpec(memory_space=pltpu.VMEM),
        scratch_shapes=[
            pltpu.VMEM((bh, s, d), jnp.bfloat16),
            pltpu.VMEM((bh, s, d), jnp.bfloat16),
            pltpu.VMEM((bh, s), jnp.float32),
            pltpu.VMEM((bh, s, d), jnp.float32),
            pltpu.SemaphoreType.DMA((8,)),
            pltpu.SemaphoreType.DMA((8,)),
        ],
        compiler_params=pltpu.CompilerParams(collective_id=0),
    )(Qb, Kb, Vb)

    return jnp.transpose(jnp.reshape(out, (b, h, s, d)), (0, 2, 1, 3))
